# baseline (speedup 1.0000x reference)
import functools

import jax
import jax.numpy as jnp
from jax import lax
from jax.experimental import pallas as pl
from jax.experimental.pallas import tpu as pltpu
from jax.experimental.pallas import tpu_sc as plsc

_T = 2
_D = 128
_NW = 32
_C = 160
_NBUF = 2
_PD = _NBUF - 1


def _sc_body(hm_hbm, h0_hbm, out_hbm, bufs, h0vs, sins, souts):
    n = hm_hbm.shape[0]
    nchunks = n // _C
    nfull = nchunks // _NW
    nrem = nchunks - nfull * _NW
    nc = 2
    wid = lax.axis_index("s") * nc + lax.axis_index("c")
    nj = jnp.where(wid < nrem, nfull + 1, nfull)

    def start_in(j, s):
        b = (wid + j * _NW) * _C
        pltpu.async_copy(hm_hbm.at[pl.ds(b, _C)], bufs[s], sins[s])
        pltpu.async_copy(h0_hbm.at[pl.ds(b, _C)], h0vs[s], sins[s])

    def wait_in(s):
        pltpu.make_async_copy(hm_hbm.at[pl.ds(0, _C)], bufs[s], sins[s]).wait()
        pltpu.make_async_copy(h0_hbm.at[pl.ds(0, _C)], h0vs[s], sins[s]).wait()

    def start_out(j, s):
        b = (wid + j * _NW) * _C
        pltpu.async_copy(bufs[s], out_hbm.at[pl.ds(b, _C)], souts[s])

    def wait_out(s):
        pltpu.make_async_copy(bufs[s], out_hbm.at[pl.ds(0, _C)], souts[s]).wait()

    for k in range(_PD):
        @pl.when(nj > k)
        def _(k=k):
            start_in(k, k)

    def step(j, slot):
        buf = bufs[slot]
        h0v = h0vs[slot]
        wait_in(slot)

        @plsc.parallel_loop(0, _C, unroll=2)
        def _row(r):
            for k in range(_D // 16):
                sl = pl.ds(k * 16, 16)
                hv = h0v[r, sl]
                plsc.addupdate(buf.at[r, 0, sl], hv)
                plsc.addupdate(buf.at[r, 1, sl], hv)
        start_out(j, slot)

        nxt = j + _PD
        nxt_slot = (slot + _PD) % _NBUF

        @pl.when(nxt < nj)
        def _():
            @pl.when(j >= _NBUF - _PD)
            def _():
                wait_out(nxt_slot)

            start_in(nxt, nxt_slot)

    def outer(jo, _):
        for s in range(_NBUF):
            j = jo * _NBUF + s

            @pl.when(j < nj)
            def _():
                step(j, s)

        return 0

    max_nj = nfull + (1 if nrem else 0)
    lax.fori_loop(0, (max_nj + _NBUF - 1) // _NBUF, outer, 0)

    for back in range(1, _NBUF + 1):
        for s in range(_NBUF):
            @pl.when(jnp.logical_and(nj >= back, (nj - back) % _NBUF == s))
            def _():
                wait_out(s)


def kernel(hm, h0):
    n, t, d = hm.shape
    mesh = plsc.VectorSubcoreMesh(core_axis_name="c", subcore_axis_name="s")
    f = functools.partial(
        pl.kernel,
        mesh=mesh,
        out_type=jax.ShapeDtypeStruct((n, t, d), jnp.float32),
        scratch_types=[
            [pltpu.VMEM((_C, _T, _D), jnp.float32) for _ in range(_NBUF)],
            [pltpu.VMEM((_C, _D), jnp.float32) for _ in range(_NBUF)],
            [pltpu.SemaphoreType.DMA for _ in range(_NBUF)],
            [pltpu.SemaphoreType.DMA for _ in range(_NBUF)],
        ],
    )(_sc_body)
    return f(hm, h0)


# speedup vs baseline: 1.4312x; 1.4312x over previous
import functools

import jax
import jax.numpy as jnp
from jax import lax
from jax.experimental import pallas as pl
from jax.experimental.pallas import tpu as pltpu
from jax.experimental.pallas import tpu_sc as plsc

_T = 2
_D = 128
_NW = 32
_C = 80
_NBUF = 4
_PD = _NBUF - 1


def _sc_body(hm_hbm, h0_hbm, out_hbm, bufs, h0vs, sins, souts):
    n = hm_hbm.shape[0]
    nchunks = n // _C
    nfull = nchunks // _NW
    nrem = nchunks - nfull * _NW
    nc = 2
    wid = lax.axis_index("s") * nc + lax.axis_index("c")
    nj = jnp.where(wid < nrem, nfull + 1, nfull)

    def start_in(j, s):
        b = (wid + j * _NW) * _C
        pltpu.async_copy(hm_hbm.at[pl.ds(b, _C)], bufs[s], sins[s])
        pltpu.async_copy(h0_hbm.at[pl.ds(b, _C)], h0vs[s], sins[s])

    def wait_in(s):
        pltpu.make_async_copy(hm_hbm.at[pl.ds(0, _C)], bufs[s], sins[s]).wait()
        pltpu.make_async_copy(h0_hbm.at[pl.ds(0, _C)], h0vs[s], sins[s]).wait()

    def start_out(j, s):
        b = (wid + j * _NW) * _C
        pltpu.async_copy(bufs[s], out_hbm.at[pl.ds(b, _C)], souts[s])

    def wait_out(s):
        pltpu.make_async_copy(bufs[s], out_hbm.at[pl.ds(0, _C)], souts[s]).wait()

    for k in range(_PD):
        @pl.when(nj > k)
        def _(k=k):
            start_in(k, k)

    def step(j, slot):
        buf = bufs[slot]
        h0v = h0vs[slot]
        wait_in(slot)

        @plsc.parallel_loop(0, _C, unroll=4)
        def _row(r):
            for k in range(_D // 16):
                sl = pl.ds(k * 16, 16)
                hv = h0v[r, sl]
                plsc.addupdate(buf.at[r, 0, sl], hv)
                plsc.addupdate(buf.at[r, 1, sl], hv)
        start_out(j, slot)

        nxt = j + _PD
        nxt_slot = (slot + _PD) % _NBUF

        @pl.when(nxt < nj)
        def _():
            @pl.when(j >= _NBUF - _PD)
            def _():
                wait_out(nxt_slot)

            start_in(nxt, nxt_slot)

    def outer(jo, _):
        for s in range(_NBUF):
            j = jo * _NBUF + s

            @pl.when(j < nj)
            def _():
                step(j, s)

        return 0

    max_nj = nfull + (1 if nrem else 0)
    lax.fori_loop(0, (max_nj + _NBUF - 1) // _NBUF, outer, 0)

    for back in range(1, _NBUF + 1):
        for s in range(_NBUF):
            @pl.when(jnp.logical_and(nj >= back, (nj - back) % _NBUF == s))
            def _():
                wait_out(s)


def kernel(hm, h0):
    n, t, d = hm.shape
    mesh = plsc.VectorSubcoreMesh(core_axis_name="c", subcore_axis_name="s")
    f = functools.partial(
        pl.kernel,
        mesh=mesh,
        out_type=jax.ShapeDtypeStruct((n, t, d), jnp.float32),
        scratch_types=[
            [pltpu.VMEM((_C, _T, _D), jnp.float32) for _ in range(_NBUF)],
            [pltpu.VMEM((_C, _D), jnp.float32) for _ in range(_NBUF)],
            [pltpu.SemaphoreType.DMA for _ in range(_NBUF)],
            [pltpu.SemaphoreType.DMA for _ in range(_NBUF)],
        ],
    )(_sc_body)
    return f(hm, h0)


# speedup vs baseline: 1.5053x; 1.0518x over previous
import functools

import jax
import jax.numpy as jnp
from jax import lax
from jax.experimental import pallas as pl
from jax.experimental.pallas import tpu as pltpu
from jax.experimental.pallas import tpu_sc as plsc

_T = 2
_D = 128
_NW = 32
_NS = 16
_C = 40
_NBUF = 4
_PD = _NBUF - 1
_NSP = 2


def _sc_body(hm_hbm, h0_hbm, out_hbm, bufs, h0vs, sp, sins, stgs, souts):
    n = hm_hbm.shape[0]
    nchunks = n // _C
    nfull = nchunks // _NW
    nrem = nchunks - nfull * _NW
    nc = 2
    sid = lax.axis_index("s")
    wid = sid * nc + lax.axis_index("c")
    nj = jnp.where(wid < nrem, nfull + 1, nfull)

    def start_in(j, s):
        b = (wid + j * _NW) * _C
        pltpu.async_copy(hm_hbm.at[pl.ds(b, _C)], bufs[s], sins[s])
        pltpu.async_copy(h0_hbm.at[pl.ds(b, _C)], h0vs[s], sins[s])

    def wait_in(s):
        pltpu.make_async_copy(hm_hbm.at[pl.ds(0, _C)], bufs[s], sins[s]).wait()
        pltpu.make_async_copy(h0_hbm.at[pl.ds(0, _C)], h0vs[s], sins[s]).wait()

    def start_stage(s, os):
        pltpu.async_copy(bufs[s], sp.at[sid, os], stgs[os])

    def wait_stage(s, os):
        pltpu.make_async_copy(bufs[s], sp.at[sid, os], stgs[os]).wait()

    def start_spout(j, os):
        b = (wid + j * _NW) * _C
        pltpu.async_copy(sp.at[sid, os], out_hbm.at[pl.ds(b, _C)], souts[os])

    def wait_spout(os):
        pltpu.make_async_copy(sp.at[sid, os], out_hbm.at[pl.ds(0, _C)],
                              souts[os]).wait()

    for k in range(_PD):
        @pl.when(nj > k)
        def _(k=k):
            start_in(k, k)

    def step(j, slot):
        buf = bufs[slot]
        h0v = h0vs[slot]
        oslot = slot % _NSP
        prev_oslot = (oslot + _NSP - 1) % _NSP
        wait_in(slot)

        @plsc.parallel_loop(0, _C, unroll=2)
        def _row(r):
            for k in range(_D // 16):
                sl = pl.ds(k * 16, 16)
                hv = h0v[r, sl]
                plsc.addupdate(buf.at[r, 0, sl], hv)
                plsc.addupdate(buf.at[r, 1, sl], hv)

        @pl.when(j >= 1)
        def _():
            wait_stage((slot + _NBUF - 1) % _NBUF, prev_oslot)
            start_spout(j - 1, prev_oslot)

        @pl.when(j >= _NSP)
        def _():
            wait_spout(oslot)

        start_stage(slot, oslot)

        nxt = j + _PD
        nxt_slot = (slot + _PD) % _NBUF

        @pl.when(nxt < nj)
        def _():
            start_in(nxt, nxt_slot)

    def outer(jo, _):
        for s in range(_NBUF):
            j = jo * _NBUF + s

            @pl.when(j < nj)
            def _():
                step(j, s)

        return 0

    max_nj = nfull + (1 if nrem else 0)
    lax.fori_loop(0, (max_nj + _NBUF - 1) // _NBUF, outer, 0)

    for os in range(_NSP):
        @pl.when(jnp.logical_and(nj >= 1, (nj - 1) % _NSP == os))
        def _(os=os):
            for s in range(_NBUF):
                @pl.when((nj - 1) % _NBUF == s)
                def _(s=s):
                    wait_stage(s, os)

            start_spout(nj - 1, os)

    for back in range(1, _NSP + 1):
        for os in range(_NSP):
            @pl.when(jnp.logical_and(nj >= back, (nj - back) % _NSP == os))
            def _(os=os):
                wait_spout(os)


def kernel(hm, h0):
    n, t, d = hm.shape
    mesh = plsc.VectorSubcoreMesh(core_axis_name="c", subcore_axis_name="s")
    f = functools.partial(
        pl.kernel,
        mesh=mesh,
        out_type=jax.ShapeDtypeStruct((n, t, d), jnp.float32),
        scratch_types=[
            [pltpu.VMEM((_C, _T, _D), jnp.float32) for _ in range(_NBUF)],
            [pltpu.VMEM((_C, _D), jnp.float32) for _ in range(_NBUF)],
            pltpu.VMEM_SHARED((_NS, _NSP, _C, _T, _D), jnp.float32),
            [pltpu.SemaphoreType.DMA for _ in range(_NBUF)],
            [pltpu.SemaphoreType.DMA for _ in range(_NSP)],
            [pltpu.SemaphoreType.DMA for _ in range(_NSP)],
        ],
    )(_sc_body)
    return f(hm, h0)
